# Initial kernel scaffold; baseline (speedup 1.0000x reference)
import jax
import jax.numpy as jnp
from jax.experimental import pallas as pl


def kernel(x, edge_index, W1, b1, W2, b2, a_prelu, Wm, bm):
    raise NotImplementedError("write your pallas kernel here")


# speedup vs baseline: 2.6982x; 2.6982x over previous
import functools

import jax
import jax.numpy as jnp
from jax import lax
from jax.experimental import pallas as pl
from jax.experimental.pallas import tpu as pltpu
from jax.experimental.pallas import tpu_sc as plsc

N = 10000
E = 320000
D = 128

NC = 2
NS = 16
NW = NC * NS
NPAD = 10240
ROWS_PER_TILE = NPAD // NS
CH = 128
CHUNKS_PER_TILE = 160
EPT = CHUNKS_PER_TILE * CH
EPAD = NS * EPT
PERM_CH = 64
PERM_CHUNKS = NPAD // (NW * PERM_CH)

_mesh = plsc.VectorSubcoreMesh(
    core_axis_name="c", subcore_axis_name="s", num_cores=NC, num_subcores=NS)

_f32 = jnp.float32


DEG_ROWS = NPAD // D
IDX_BLK = 2048
N_IDX_BLK = EPT // IDX_BLK


@functools.partial(
    pl.kernel,
    out_type=(
        jax.ShapeDtypeStruct((DEG_ROWS, D), _f32),
        jax.ShapeDtypeStruct((DEG_ROWS, D), _f32),
        jax.ShapeDtypeStruct((NPAD, D), _f32),
    ),
    mesh=_mesh,
    compiler_params=pltpu.CompilerParams(needs_layout_passes=False),
    scratch_types=(
        pltpu.VMEM((IDX_BLK,), jnp.int32),
        pltpu.VMEM((DEG_ROWS, D), _f32),
        pltpu.VMEM((DEG_ROWS,), jnp.int32),
        pltpu.VMEM((PERM_CHUNKS * PERM_CH,), jnp.int32),
        pltpu.VMEM((PERM_CH, D), _f32),
        pltpu.VMEM_SHARED((DEG_ROWS, D), _f32),
        pltpu.SemaphoreType.DMA,
    ),
)
def _prep(src_hbm, dst_hbm, xpad_hbm, perm_hbm, iota_hbm, zfeat_hbm,
          outdeg_hbm, indeg_hbm, xp_hbm,
          idx_v, acc_v, iota_v, pidx_v, xrows_v, deg_sh, sem):
    c = lax.axis_index("c")
    s = lax.axis_index("s")

    @pl.when(s == 0)
    def _():
        pltpu.sync_copy(zfeat_hbm.at[pl.ds(0, DEG_ROWS)], deg_sh)

    pltpu.sync_copy(zfeat_hbm.at[pl.ds(0, DEG_ROWS)], acc_v)
    pltpu.sync_copy(iota_hbm, iota_v)
    plsc.subcore_barrier()

    base = s * EPT
    ones_lane = jnp.ones((16,), _f32)

    def make_hist(edge_hbm):
        def blk_body(b, carry):
            pltpu.sync_copy(edge_hbm.at[pl.ds(base + b * IDX_BLK, IDX_BLK)],
                            idx_v)

            def lane_body(k, carry2):
                iv = idx_v[pl.ds(k * 16, 16)]
                hi = lax.shift_right_logical(iv, 7)
                lo = jnp.bitwise_and(iv, 127)
                plsc.addupdate_scatter(acc_v, [hi, lo], ones_lane)
                return carry2

            lax.fori_loop(0, IDX_BLK // 16, lane_body, 0)
            return carry
        return blk_body

    @pl.when(c == 0)
    def _():
        lax.fori_loop(0, N_IDX_BLK, make_hist(src_hbm), 0)

    @pl.when(c == 1)
    def _():
        lax.fori_loop(0, N_IDX_BLK, make_hist(dst_hbm), 0)

    pltpu.sync_copy(acc_v, deg_sh.at[iota_v], add=True)
    plsc.subcore_barrier()

    @pl.when((s == 0) & (c == 0))
    def _():
        pltpu.sync_copy(deg_sh, outdeg_hbm)

    @pl.when((s == 0) & (c == 1))
    def _():
        pltpu.sync_copy(deg_sh, indeg_hbm)

    w = s * NC + c
    pltpu.sync_copy(perm_hbm.at[pl.ds(w * PERM_CHUNKS * PERM_CH,
                                      PERM_CHUNKS * PERM_CH)], pidx_v)

    def perm_body(j, carry):
        idx = pidx_v.at[pl.ds(j * PERM_CH, PERM_CH)]
        pltpu.async_copy(xpad_hbm.at[idx], xrows_v, sem).wait()
        pltpu.sync_copy(
            xrows_v,
            xp_hbm.at[pl.ds(w * PERM_CHUNKS * PERM_CH + j * PERM_CH, PERM_CH)])
        return carry

    lax.fori_loop(0, PERM_CHUNKS, perm_body, 0)


@functools.partial(
    pl.kernel,
    out_type=(
        jax.ShapeDtypeStruct((NPAD, D), _f32),
        jax.ShapeDtypeStruct((NPAD, D), _f32),
    ),
    mesh=_mesh,
    scratch_types=(
        pltpu.VMEM((CH,), jnp.int32),
        pltpu.VMEM((CH,), jnp.int32),
        pltpu.VMEM((CH, D), _f32),
        pltpu.VMEM_SHARED((NPAD, D), _f32),
        pltpu.SemaphoreType.DMA,
    ),
)
def _scatter(ha_hbm, hb_hbm, src_hbm, dst_hbm, zfeat_hbm,
             agga_hbm, aggb_hbm,
             sidx_v, didx_v, rows_v, acc_sh, sem):
    c = lax.axis_index("c")
    s = lax.axis_index("s")

    sl = pl.ds(s * ROWS_PER_TILE, ROWS_PER_TILE)
    pltpu.sync_copy(zfeat_hbm, acc_sh.at[sl])
    plsc.subcore_barrier()
    base = s * EPT

    def make_body(h_hbm):
        def body(j, carry):
            pltpu.sync_copy(src_hbm.at[pl.ds(base + j * CH, CH)], sidx_v)
            pltpu.sync_copy(dst_hbm.at[pl.ds(base + j * CH, CH)], didx_v)
            pltpu.async_copy(h_hbm.at[sidx_v], rows_v, sem).wait()
            pltpu.sync_copy(rows_v, acc_sh.at[didx_v], add=True)
            return carry
        return body

    @pl.when(c == 0)
    def _():
        lax.fori_loop(0, CHUNKS_PER_TILE, make_body(ha_hbm), 0)

    @pl.when(c == 1)
    def _():
        lax.fori_loop(0, CHUNKS_PER_TILE, make_body(hb_hbm), 0)

    plsc.subcore_barrier()

    @pl.when(c == 0)
    def _():
        pltpu.sync_copy(acc_sh.at[sl], agga_hbm.at[sl])

    @pl.when(c == 1)
    def _():
        pltpu.sync_copy(acc_sh.at[sl], aggb_hbm.at[sl])


def _prelu(v, a):
    return jnp.where(v >= 0, v, v * a)


def _tc1_body(x_ref, xp_ref, od_ref, w1_ref, oa_ref, ob_ref):
    on = lax.rsqrt(jnp.maximum(od_ref[...], 1.0))
    w = w1_ref[...]
    oa_ref[...] = jnp.dot(x_ref[...] * on, w, preferred_element_type=_f32,
                         precision=lax.Precision.HIGHEST)
    ob_ref[...] = jnp.dot(xp_ref[...] * on, w, preferred_element_type=_f32,
                         precision=lax.Precision.HIGHEST)


def _tc2_body(aa_ref, ab_ref, id_ref, od_ref, b_ref, a_ref, w2_ref,
              oa_ref, ob_ref):
    inn = lax.rsqrt(jnp.maximum(id_ref[...], 1.0))
    on = lax.rsqrt(jnp.maximum(od_ref[...], 1.0))
    a = a_ref[...]
    b = b_ref[...]
    w = w2_ref[...]
    za = _prelu(aa_ref[...] * inn + b, a)
    zb = _prelu(ab_ref[...] * inn + b, a)
    oa_ref[...] = jnp.dot(za * on, w, preferred_element_type=_f32,
                         precision=lax.Precision.HIGHEST)
    ob_ref[...] = jnp.dot(zb * on, w, preferred_element_type=_f32,
                         precision=lax.Precision.HIGHEST)


def _tc3_body(aa_ref, ab_ref, id_ref, b_ref, a_ref, wm_ref, bm_ref,
              oa_ref, ob_ref):
    inn = lax.rsqrt(jnp.maximum(id_ref[...], 1.0))
    a = a_ref[...]
    b = b_ref[...]
    za = _prelu(aa_ref[...] * inn + b, a)
    zb = _prelu(ab_ref[...] * inn + b, a)
    wv = jnp.sum(wm_ref[...], axis=1, keepdims=True)
    bs = jnp.sum(bm_ref[...])
    oa_ref[...] = jnp.dot(za, wv, preferred_element_type=_f32,
                         precision=lax.Precision.HIGHEST) + bs
    ob_ref[...] = jnp.dot(zb, wv, preferred_element_type=_f32,
                         precision=lax.Precision.HIGHEST) + bs


_TCB = 2560
_GRID = (NPAD // _TCB,)


def _row_spec(width):
    return pl.BlockSpec((_TCB, width), lambda i: (i, 0))


def _fix_spec(r, cdim):
    return pl.BlockSpec((r, cdim), lambda i: (0, 0))


_tc1 = pl.pallas_call(
    _tc1_body,
    grid=_GRID,
    in_specs=[_row_spec(D), _row_spec(D), _row_spec(1), _fix_spec(D, D)],
    out_specs=(_row_spec(D), _row_spec(D)),
    out_shape=(jax.ShapeDtypeStruct((NPAD, D), _f32),
               jax.ShapeDtypeStruct((NPAD, D), _f32)))

_tc2 = pl.pallas_call(
    _tc2_body,
    grid=_GRID,
    in_specs=[_row_spec(D), _row_spec(D), _row_spec(1), _row_spec(1),
              _fix_spec(1, D), _fix_spec(1, D), _fix_spec(D, D)],
    out_specs=(_row_spec(D), _row_spec(D)),
    out_shape=(jax.ShapeDtypeStruct((NPAD, D), _f32),
               jax.ShapeDtypeStruct((NPAD, D), _f32)))

_tc3 = pl.pallas_call(
    _tc3_body,
    grid=_GRID,
    in_specs=[_row_spec(D), _row_spec(D), _row_spec(1),
              _fix_spec(1, D), _fix_spec(1, D), _fix_spec(D, D),
              _fix_spec(1, D)],
    out_specs=(_row_spec(1), _row_spec(1)),
    out_shape=(jax.ShapeDtypeStruct((NPAD, 1), _f32),
               jax.ShapeDtypeStruct((NPAD, 1), _f32)))


def kernel(x, edge_index, W1, b1, W2, b2, a_prelu, Wm, bm):
    src = edge_index[0].astype(jnp.int32)
    dst = edge_index[1].astype(jnp.int32)
    pad = jnp.full((EPAD - E,), N, dtype=jnp.int32)
    src1d = jnp.concatenate([src, pad])
    dst1d = jnp.concatenate([dst, pad])

    xpad = jnp.pad(x, ((0, NPAD - N), (0, 0)))
    perm = jax.random.permutation(jax.random.key(1), N).astype(jnp.int32)
    perm1d = jnp.concatenate([perm, jnp.full((NPAD - N,), N, dtype=jnp.int32)])

    iota80 = jnp.arange(DEG_ROWS, dtype=jnp.int32)
    zfeat = jnp.zeros((ROWS_PER_TILE, D), dtype=_f32)

    outdeg_g, indeg_g, xp = _prep(src1d, dst1d, xpad, perm1d, iota80, zfeat)
    outdeg = outdeg_g.reshape(NPAD, 1)
    indeg = indeg_g.reshape(NPAD, 1)

    b1r = b1.reshape(1, D)
    b2r = b2.reshape(1, D)
    ar = a_prelu.reshape(1, D)
    bmr = bm.reshape(1, D)

    ha1, hb1 = _tc1(xpad, xp, outdeg, W1)
    agga1, aggb1 = _scatter(ha1, hb1, src1d, dst1d, zfeat)
    ha2, hb2 = _tc2(agga1, aggb1, indeg, outdeg, b1r, ar, W2)
    agga2, aggb2 = _scatter(ha2, hb2, src1d, dst1d, zfeat)
    oa, ob = _tc3(agga2, aggb2, indeg, b2r, ar, Wm, bmr)

    return jnp.concatenate([oa[:N, 0], ob[:N, 0]], axis=0)


# speedup vs baseline: 3.8152x; 1.4140x over previous
import functools

import jax
import jax.numpy as jnp
from jax import lax
from jax.experimental import pallas as pl
from jax.experimental.pallas import tpu as pltpu
from jax.experimental.pallas import tpu_sc as plsc

N = 10000
E = 320000
D = 128

NC = 2
NS = 16
NW = NC * NS
NPAD = 10240
ROWS_PER_TILE = NPAD // NS
CH = 128
CHUNKS_PER_TILE = 160
EPT = CHUNKS_PER_TILE * CH
EPAD = NS * EPT
PERM_CH = 64
PERM_CHUNKS = NPAD // (NW * PERM_CH)

_mesh = plsc.VectorSubcoreMesh(
    core_axis_name="c", subcore_axis_name="s", num_cores=NC, num_subcores=NS)

_f32 = jnp.float32


DEG_ROWS = NPAD // D
IDX_BLK = 2048
N_IDX_BLK = EPT // IDX_BLK


@functools.partial(
    pl.kernel,
    out_type=(
        jax.ShapeDtypeStruct((DEG_ROWS, D), _f32),
        jax.ShapeDtypeStruct((DEG_ROWS, D), _f32),
        jax.ShapeDtypeStruct((NPAD, D), _f32),
    ),
    mesh=_mesh,
    compiler_params=pltpu.CompilerParams(needs_layout_passes=False),
    scratch_types=(
        pltpu.VMEM((IDX_BLK,), jnp.int32),
        pltpu.VMEM((DEG_ROWS, D), _f32),
        pltpu.VMEM((DEG_ROWS,), jnp.int32),
        pltpu.VMEM((PERM_CHUNKS * PERM_CH,), jnp.int32),
        pltpu.VMEM((PERM_CH, D), _f32),
        pltpu.VMEM_SHARED((DEG_ROWS, D), _f32),
        pltpu.SemaphoreType.DMA,
    ),
)
def _prep(src_hbm, dst_hbm, xpad_hbm, perm_hbm, iota_hbm, zfeat_hbm,
          outdeg_hbm, indeg_hbm, xp_hbm,
          idx_v, acc_v, iota_v, pidx_v, xrows_v, deg_sh, sem):
    c = lax.axis_index("c")
    s = lax.axis_index("s")

    @pl.when(s == 0)
    def _():
        pltpu.sync_copy(zfeat_hbm.at[pl.ds(0, DEG_ROWS)], deg_sh)

    pltpu.sync_copy(zfeat_hbm.at[pl.ds(0, DEG_ROWS)], acc_v)
    pltpu.sync_copy(iota_hbm, iota_v)
    plsc.subcore_barrier()

    base = s * EPT
    ones_lane = jnp.ones((16,), _f32)

    def make_hist(edge_hbm):
        def blk_body(b, carry):
            pltpu.sync_copy(edge_hbm.at[pl.ds(base + b * IDX_BLK, IDX_BLK)],
                            idx_v)

            def lane_body(k, carry2):
                iv = idx_v[pl.ds(k * 16, 16)]
                hi = lax.shift_right_logical(iv, 7)
                lo = jnp.bitwise_and(iv, 127)
                plsc.addupdate_scatter(acc_v, [hi, lo], ones_lane)
                return carry2

            lax.fori_loop(0, IDX_BLK // 16, lane_body, 0)
            return carry
        return blk_body

    @pl.when(c == 0)
    def _():
        lax.fori_loop(0, N_IDX_BLK, make_hist(src_hbm), 0)

    @pl.when(c == 1)
    def _():
        lax.fori_loop(0, N_IDX_BLK, make_hist(dst_hbm), 0)

    pltpu.sync_copy(acc_v, deg_sh.at[iota_v], add=True)
    plsc.subcore_barrier()

    @pl.when((s == 0) & (c == 0))
    def _():
        pltpu.sync_copy(deg_sh, outdeg_hbm)

    @pl.when((s == 0) & (c == 1))
    def _():
        pltpu.sync_copy(deg_sh, indeg_hbm)

    w = s * NC + c
    pltpu.sync_copy(perm_hbm.at[pl.ds(w * PERM_CHUNKS * PERM_CH,
                                      PERM_CHUNKS * PERM_CH)], pidx_v)

    def perm_body(j, carry):
        idx = pidx_v.at[pl.ds(j * PERM_CH, PERM_CH)]
        pltpu.async_copy(xpad_hbm.at[idx], xrows_v, sem).wait()
        pltpu.sync_copy(
            xrows_v,
            xp_hbm.at[pl.ds(w * PERM_CHUNKS * PERM_CH + j * PERM_CH, PERM_CH)])
        return carry

    lax.fori_loop(0, PERM_CHUNKS, perm_body, 0)


CHUNKS_PER_BLK = IDX_BLK // CH


@functools.partial(
    pl.kernel,
    out_type=(
        jax.ShapeDtypeStruct((NPAD, D), _f32),
        jax.ShapeDtypeStruct((NPAD, D), _f32),
    ),
    mesh=_mesh,
    scratch_types=(
        pltpu.VMEM((IDX_BLK,), jnp.int32),
        pltpu.VMEM((IDX_BLK,), jnp.int32),
        pltpu.VMEM((CH, D), _f32),
        pltpu.VMEM((CH, D), _f32),
        pltpu.VMEM_SHARED((NPAD, D), _f32),
        pltpu.SemaphoreType.DMA,
        pltpu.SemaphoreType.DMA,
    ),
)
def _scatter(ha_hbm, hb_hbm, src_hbm, dst_hbm, zfeat_hbm,
             agga_hbm, aggb_hbm,
             sidx_v, didx_v, rows0_v, rows1_v, acc_sh, gsem0, gsem1):
    c = lax.axis_index("c")
    s = lax.axis_index("s")

    sl = pl.ds(s * ROWS_PER_TILE, ROWS_PER_TILE)
    pltpu.sync_copy(zfeat_hbm, acc_sh.at[sl])
    plsc.subcore_barrier()
    base = s * EPT

    def make_pass(h_hbm):
        def fire(k, rows_v, gsem):
            pltpu.async_copy(h_hbm.at[sidx_v.at[pl.ds(k * CH, CH)]],
                             rows_v, gsem)

        def gwait(rows_v, gsem):
            pltpu.make_async_copy(h_hbm.at[pl.ds(0, CH)], rows_v, gsem).wait()

        def scat(k, rows_v):
            pltpu.sync_copy(rows_v, acc_sh.at[didx_v.at[pl.ds(k * CH, CH)]],
                            add=True)

        def blk_body(b, carry):
            off = base + b * IDX_BLK
            pltpu.sync_copy(src_hbm.at[pl.ds(off, IDX_BLK)], sidx_v)
            pltpu.sync_copy(dst_hbm.at[pl.ds(off, IDX_BLK)], didx_v)
            fire(0, rows0_v, gsem0)

            def m_body(m, carry2):
                j0 = 2 * m
                fire(j0 + 1, rows1_v, gsem1)
                gwait(rows0_v, gsem0)
                scat(j0, rows0_v)

                @pl.when(m < CHUNKS_PER_BLK // 2 - 1)
                def _():
                    fire(j0 + 2, rows0_v, gsem0)

                gwait(rows1_v, gsem1)
                scat(j0 + 1, rows1_v)
                return carry2

            lax.fori_loop(0, CHUNKS_PER_BLK // 2, m_body, 0)
            return carry
        return blk_body

    @pl.when(c == 0)
    def _():
        lax.fori_loop(0, N_IDX_BLK, make_pass(ha_hbm), 0)

    @pl.when(c == 1)
    def _():
        lax.fori_loop(0, N_IDX_BLK, make_pass(hb_hbm), 0)

    plsc.subcore_barrier()

    @pl.when(c == 0)
    def _():
        pltpu.sync_copy(acc_sh.at[sl], agga_hbm.at[sl])

    @pl.when(c == 1)
    def _():
        pltpu.sync_copy(acc_sh.at[sl], aggb_hbm.at[sl])


def _prelu(v, a):
    return jnp.where(v >= 0, v, v * a)


def _tc1_body(x_ref, xp_ref, od_ref, w1_ref, oa_ref, ob_ref):
    on = lax.rsqrt(jnp.maximum(od_ref[...], 1.0))
    w = w1_ref[...]
    oa_ref[...] = jnp.dot(x_ref[...] * on, w, preferred_element_type=_f32,
                         precision=lax.Precision.HIGHEST)
    ob_ref[...] = jnp.dot(xp_ref[...] * on, w, preferred_element_type=_f32,
                         precision=lax.Precision.HIGHEST)


def _tc2_body(aa_ref, ab_ref, id_ref, od_ref, b_ref, a_ref, w2_ref,
              oa_ref, ob_ref):
    inn = lax.rsqrt(jnp.maximum(id_ref[...], 1.0))
    on = lax.rsqrt(jnp.maximum(od_ref[...], 1.0))
    a = a_ref[...]
    b = b_ref[...]
    w = w2_ref[...]
    za = _prelu(aa_ref[...] * inn + b, a)
    zb = _prelu(ab_ref[...] * inn + b, a)
    oa_ref[...] = jnp.dot(za * on, w, preferred_element_type=_f32,
                         precision=lax.Precision.HIGHEST)
    ob_ref[...] = jnp.dot(zb * on, w, preferred_element_type=_f32,
                         precision=lax.Precision.HIGHEST)


def _tc3_body(aa_ref, ab_ref, id_ref, b_ref, a_ref, wm_ref, bm_ref,
              oa_ref, ob_ref):
    inn = lax.rsqrt(jnp.maximum(id_ref[...], 1.0))
    a = a_ref[...]
    b = b_ref[...]
    za = _prelu(aa_ref[...] * inn + b, a)
    zb = _prelu(ab_ref[...] * inn + b, a)
    wv = jnp.sum(wm_ref[...], axis=1, keepdims=True)
    bs = jnp.sum(bm_ref[...])
    oa_ref[...] = jnp.dot(za, wv, preferred_element_type=_f32,
                         precision=lax.Precision.HIGHEST) + bs
    ob_ref[...] = jnp.dot(zb, wv, preferred_element_type=_f32,
                         precision=lax.Precision.HIGHEST) + bs


_TCB = 2560
_GRID = (NPAD // _TCB,)


def _row_spec(width):
    return pl.BlockSpec((_TCB, width), lambda i: (i, 0))


def _fix_spec(r, cdim):
    return pl.BlockSpec((r, cdim), lambda i: (0, 0))


_tc1 = pl.pallas_call(
    _tc1_body,
    grid=_GRID,
    in_specs=[_row_spec(D), _row_spec(D), _row_spec(1), _fix_spec(D, D)],
    out_specs=(_row_spec(D), _row_spec(D)),
    out_shape=(jax.ShapeDtypeStruct((NPAD, D), _f32),
               jax.ShapeDtypeStruct((NPAD, D), _f32)))

_tc2 = pl.pallas_call(
    _tc2_body,
    grid=_GRID,
    in_specs=[_row_spec(D), _row_spec(D), _row_spec(1), _row_spec(1),
              _fix_spec(1, D), _fix_spec(1, D), _fix_spec(D, D)],
    out_specs=(_row_spec(D), _row_spec(D)),
    out_shape=(jax.ShapeDtypeStruct((NPAD, D), _f32),
               jax.ShapeDtypeStruct((NPAD, D), _f32)))

_tc3 = pl.pallas_call(
    _tc3_body,
    grid=_GRID,
    in_specs=[_row_spec(D), _row_spec(D), _row_spec(1),
              _fix_spec(1, D), _fix_spec(1, D), _fix_spec(D, D),
              _fix_spec(1, D)],
    out_specs=(_row_spec(1), _row_spec(1)),
    out_shape=(jax.ShapeDtypeStruct((NPAD, 1), _f32),
               jax.ShapeDtypeStruct((NPAD, 1), _f32)))


def kernel(x, edge_index, W1, b1, W2, b2, a_prelu, Wm, bm):
    src = edge_index[0].astype(jnp.int32)
    dst = edge_index[1].astype(jnp.int32)
    pad = jnp.full((EPAD - E,), N, dtype=jnp.int32)
    src1d = jnp.concatenate([src, pad])
    dst1d = jnp.concatenate([dst, pad])

    xpad = jnp.pad(x, ((0, NPAD - N), (0, 0)))
    perm = jax.random.permutation(jax.random.key(1), N).astype(jnp.int32)
    perm1d = jnp.concatenate([perm, jnp.full((NPAD - N,), N, dtype=jnp.int32)])

    iota80 = jnp.arange(DEG_ROWS, dtype=jnp.int32)
    zfeat = jnp.zeros((ROWS_PER_TILE, D), dtype=_f32)

    outdeg_g, indeg_g, xp = _prep(src1d, dst1d, xpad, perm1d, iota80, zfeat)
    outdeg = outdeg_g.reshape(NPAD, 1)
    indeg = indeg_g.reshape(NPAD, 1)

    b1r = b1.reshape(1, D)
    b2r = b2.reshape(1, D)
    ar = a_prelu.reshape(1, D)
    bmr = bm.reshape(1, D)

    ha1, hb1 = _tc1(xpad, xp, outdeg, W1)
    agga1, aggb1 = _scatter(ha1, hb1, src1d, dst1d, zfeat)
    ha2, hb2 = _tc2(agga1, aggb1, indeg, outdeg, b1r, ar, W2)
    agga2, aggb2 = _scatter(ha2, hb2, src1d, dst1d, zfeat)
    oa, ob = _tc3(agga2, aggb2, indeg, b2r, ar, Wm, bmr)

    return jnp.concatenate([oa[:N, 0], ob[:N, 0]], axis=0)


# speedup vs baseline: 3.8634x; 1.0126x over previous
import functools

import jax
import jax.numpy as jnp
from jax import lax
from jax.experimental import pallas as pl
from jax.experimental.pallas import tpu as pltpu
from jax.experimental.pallas import tpu_sc as plsc

N = 10000
E = 320000
D = 128

NC = 2
NS = 16
NW = NC * NS
NPAD = 10240
ROWS_PER_TILE = NPAD // NS
CH = 128
CHUNKS_PER_TILE = 160
EPT = CHUNKS_PER_TILE * CH
EPAD = NS * EPT
PERM_CH = 64
PERM_CHUNKS = NPAD // (NW * PERM_CH)

_mesh = plsc.VectorSubcoreMesh(
    core_axis_name="c", subcore_axis_name="s", num_cores=NC, num_subcores=NS)

_f32 = jnp.float32


DEG_ROWS = NPAD // D
IDX_BLK = 2048
N_IDX_BLK = EPT // IDX_BLK


@functools.partial(
    pl.kernel,
    out_type=(
        jax.ShapeDtypeStruct((DEG_ROWS, D), _f32),
        jax.ShapeDtypeStruct((DEG_ROWS, D), _f32),
        jax.ShapeDtypeStruct((NPAD, D), _f32),
    ),
    mesh=_mesh,
    compiler_params=pltpu.CompilerParams(needs_layout_passes=False),
    scratch_types=(
        pltpu.VMEM((IDX_BLK,), jnp.int32),
        pltpu.VMEM((DEG_ROWS, D), _f32),
        pltpu.VMEM((DEG_ROWS,), jnp.int32),
        pltpu.VMEM((PERM_CHUNKS * PERM_CH,), jnp.int32),
        pltpu.VMEM((PERM_CH, D), _f32),
        pltpu.VMEM_SHARED((DEG_ROWS, D), _f32),
        pltpu.SemaphoreType.DMA,
    ),
)
def _prep(src_hbm, dst_hbm, xpad_hbm, perm_hbm, iota_hbm, zfeat_hbm,
          outdeg_hbm, indeg_hbm, xp_hbm,
          idx_v, acc_v, iota_v, pidx_v, xrows_v, deg_sh, sem):
    c = lax.axis_index("c")
    s = lax.axis_index("s")

    @pl.when(s == 0)
    def _():
        pltpu.sync_copy(zfeat_hbm.at[pl.ds(0, DEG_ROWS)], deg_sh)

    pltpu.sync_copy(zfeat_hbm.at[pl.ds(0, DEG_ROWS)], acc_v)
    pltpu.sync_copy(iota_hbm, iota_v)
    plsc.subcore_barrier()

    base = s * EPT
    ones_lane = jnp.ones((16,), _f32)

    def make_hist(edge_hbm):
        def blk_body(b, carry):
            pltpu.sync_copy(edge_hbm.at[pl.ds(base + b * IDX_BLK, IDX_BLK)],
                            idx_v)

            def lane_body(k, carry2):
                iv = idx_v[pl.ds(k * 16, 16)]
                hi = lax.shift_right_logical(iv, 7)
                lo = jnp.bitwise_and(iv, 127)
                plsc.addupdate_scatter(acc_v, [hi, lo], ones_lane)
                return carry2

            lax.fori_loop(0, IDX_BLK // 16, lane_body, 0)
            return carry
        return blk_body

    @pl.when(c == 0)
    def _():
        lax.fori_loop(0, N_IDX_BLK, make_hist(src_hbm), 0)

    @pl.when(c == 1)
    def _():
        lax.fori_loop(0, N_IDX_BLK, make_hist(dst_hbm), 0)

    pltpu.sync_copy(acc_v, deg_sh.at[iota_v], add=True)
    plsc.subcore_barrier()

    @pl.when((s == 0) & (c == 0))
    def _():
        pltpu.sync_copy(deg_sh, outdeg_hbm)

    @pl.when((s == 0) & (c == 1))
    def _():
        pltpu.sync_copy(deg_sh, indeg_hbm)

    w = s * NC + c
    pltpu.sync_copy(perm_hbm.at[pl.ds(w * PERM_CHUNKS * PERM_CH,
                                      PERM_CHUNKS * PERM_CH)], pidx_v)

    def perm_body(j, carry):
        idx = pidx_v.at[pl.ds(j * PERM_CH, PERM_CH)]
        pltpu.async_copy(xpad_hbm.at[idx], xrows_v, sem).wait()
        pltpu.sync_copy(
            xrows_v,
            xp_hbm.at[pl.ds(w * PERM_CHUNKS * PERM_CH + j * PERM_CH, PERM_CH)])
        return carry

    lax.fori_loop(0, PERM_CHUNKS, perm_body, 0)


CHUNKS_PER_BLK = IDX_BLK // CH


@functools.partial(
    pl.kernel,
    out_type=(
        jax.ShapeDtypeStruct((NPAD, D), _f32),
        jax.ShapeDtypeStruct((NPAD, D), _f32),
    ),
    mesh=_mesh,
    scratch_types=(
        pltpu.VMEM((IDX_BLK,), jnp.int32),
        pltpu.VMEM((IDX_BLK,), jnp.int32),
        pltpu.VMEM((CH, D), _f32),
        pltpu.VMEM((CH, D), _f32),
        pltpu.VMEM_SHARED((NPAD, D), _f32),
        pltpu.SemaphoreType.DMA,
        pltpu.SemaphoreType.DMA,
    ),
)
def _scatter(ha_hbm, hb_hbm, src_hbm, dst_hbm, zfeat_hbm,
             agga_hbm, aggb_hbm,
             sidx_v, didx_v, rows0_v, rows1_v, acc_sh, gsem0, gsem1):
    c = lax.axis_index("c")
    s = lax.axis_index("s")

    sl = pl.ds(s * ROWS_PER_TILE, ROWS_PER_TILE)
    pltpu.sync_copy(zfeat_hbm, acc_sh.at[sl])
    plsc.subcore_barrier()
    base = s * EPT

    def make_pass(h_hbm):
        def fire(k, rows_v, gsem):
            pltpu.async_copy(h_hbm.at[sidx_v.at[pl.ds(k * CH, CH)]],
                             rows_v, gsem)

        def gwait(rows_v, gsem):
            pltpu.make_async_copy(h_hbm.at[pl.ds(0, CH)], rows_v, gsem).wait()

        def scat(k, rows_v):
            pltpu.sync_copy(rows_v, acc_sh.at[didx_v.at[pl.ds(k * CH, CH)]],
                            add=True)

        def blk_body(b, carry):
            off = base + b * IDX_BLK
            pltpu.sync_copy(src_hbm.at[pl.ds(off, IDX_BLK)], sidx_v)
            pltpu.sync_copy(dst_hbm.at[pl.ds(off, IDX_BLK)], didx_v)
            fire(0, rows0_v, gsem0)

            def m_body(m, carry2):
                j0 = 2 * m
                fire(j0 + 1, rows1_v, gsem1)
                gwait(rows0_v, gsem0)
                scat(j0, rows0_v)

                @pl.when(m < CHUNKS_PER_BLK // 2 - 1)
                def _():
                    fire(j0 + 2, rows0_v, gsem0)

                gwait(rows1_v, gsem1)
                scat(j0 + 1, rows1_v)
                return carry2

            lax.fori_loop(0, CHUNKS_PER_BLK // 2, m_body, 0)
            return carry
        return blk_body

    @pl.when(c == 0)
    def _():
        lax.fori_loop(0, N_IDX_BLK, make_pass(ha_hbm), 0)

    @pl.when(c == 1)
    def _():
        lax.fori_loop(0, N_IDX_BLK, make_pass(hb_hbm), 0)

    plsc.subcore_barrier()

    @pl.when(c == 0)
    def _():
        pltpu.sync_copy(acc_sh.at[sl], agga_hbm.at[sl])

    @pl.when(c == 1)
    def _():
        pltpu.sync_copy(acc_sh.at[sl], aggb_hbm.at[sl])


def _prelu(v, a):
    return jnp.where(v >= 0, v, v * a)


def _tc1_body(x_ref, xp_ref, od_ref, w1_ref, oa_ref, ob_ref):
    on = lax.rsqrt(jnp.maximum(od_ref[...], 1.0))
    w = w1_ref[...]
    oa_ref[...] = jnp.dot(x_ref[...] * on, w, preferred_element_type=_f32,
                         precision=lax.Precision.DEFAULT)
    ob_ref[...] = jnp.dot(xp_ref[...] * on, w, preferred_element_type=_f32,
                         precision=lax.Precision.DEFAULT)


def _tc2_body(aa_ref, ab_ref, id_ref, od_ref, b_ref, a_ref, w2_ref,
              oa_ref, ob_ref):
    inn = lax.rsqrt(jnp.maximum(id_ref[...], 1.0))
    on = lax.rsqrt(jnp.maximum(od_ref[...], 1.0))
    a = a_ref[...]
    b = b_ref[...]
    w = w2_ref[...]
    za = _prelu(aa_ref[...] * inn + b, a)
    zb = _prelu(ab_ref[...] * inn + b, a)
    oa_ref[...] = jnp.dot(za * on, w, preferred_element_type=_f32,
                         precision=lax.Precision.DEFAULT)
    ob_ref[...] = jnp.dot(zb * on, w, preferred_element_type=_f32,
                         precision=lax.Precision.DEFAULT)


def _tc3_body(aa_ref, ab_ref, id_ref, b_ref, a_ref, wm_ref, bm_ref,
              oa_ref, ob_ref):
    inn = lax.rsqrt(jnp.maximum(id_ref[...], 1.0))
    a = a_ref[...]
    b = b_ref[...]
    za = _prelu(aa_ref[...] * inn + b, a)
    zb = _prelu(ab_ref[...] * inn + b, a)
    wv = jnp.sum(wm_ref[...], axis=1, keepdims=True)
    bs = jnp.sum(bm_ref[...])
    oa_ref[...] = jnp.dot(za, wv, preferred_element_type=_f32,
                         precision=lax.Precision.DEFAULT) + bs
    ob_ref[...] = jnp.dot(zb, wv, preferred_element_type=_f32,
                         precision=lax.Precision.DEFAULT) + bs


_TCB = 2560
_GRID = (NPAD // _TCB,)


def _row_spec(width):
    return pl.BlockSpec((_TCB, width), lambda i: (i, 0))


def _fix_spec(r, cdim):
    return pl.BlockSpec((r, cdim), lambda i: (0, 0))


_tc1 = pl.pallas_call(
    _tc1_body,
    grid=_GRID,
    in_specs=[_row_spec(D), _row_spec(D), _row_spec(1), _fix_spec(D, D)],
    out_specs=(_row_spec(D), _row_spec(D)),
    out_shape=(jax.ShapeDtypeStruct((NPAD, D), _f32),
               jax.ShapeDtypeStruct((NPAD, D), _f32)))

_tc2 = pl.pallas_call(
    _tc2_body,
    grid=_GRID,
    in_specs=[_row_spec(D), _row_spec(D), _row_spec(1), _row_spec(1),
              _fix_spec(1, D), _fix_spec(1, D), _fix_spec(D, D)],
    out_specs=(_row_spec(D), _row_spec(D)),
    out_shape=(jax.ShapeDtypeStruct((NPAD, D), _f32),
               jax.ShapeDtypeStruct((NPAD, D), _f32)))

_tc3 = pl.pallas_call(
    _tc3_body,
    grid=_GRID,
    in_specs=[_row_spec(D), _row_spec(D), _row_spec(1),
              _fix_spec(1, D), _fix_spec(1, D), _fix_spec(D, D),
              _fix_spec(1, D)],
    out_specs=(_row_spec(1), _row_spec(1)),
    out_shape=(jax.ShapeDtypeStruct((NPAD, 1), _f32),
               jax.ShapeDtypeStruct((NPAD, 1), _f32)))


def kernel(x, edge_index, W1, b1, W2, b2, a_prelu, Wm, bm):
    src = edge_index[0].astype(jnp.int32)
    dst = edge_index[1].astype(jnp.int32)
    pad = jnp.full((EPAD - E,), N, dtype=jnp.int32)
    src1d = jnp.concatenate([src, pad])
    dst1d = jnp.concatenate([dst, pad])

    xpad = jnp.pad(x, ((0, NPAD - N), (0, 0)))
    perm = jax.random.permutation(jax.random.key(1), N).astype(jnp.int32)
    perm1d = jnp.concatenate([perm, jnp.full((NPAD - N,), N, dtype=jnp.int32)])

    iota80 = jnp.arange(DEG_ROWS, dtype=jnp.int32)
    zfeat = jnp.zeros((ROWS_PER_TILE, D), dtype=_f32)

    outdeg_g, indeg_g, xp = _prep(src1d, dst1d, xpad, perm1d, iota80, zfeat)
    outdeg = outdeg_g.reshape(NPAD, 1)
    indeg = indeg_g.reshape(NPAD, 1)

    b1r = b1.reshape(1, D)
    b2r = b2.reshape(1, D)
    ar = a_prelu.reshape(1, D)
    bmr = bm.reshape(1, D)

    ha1, hb1 = _tc1(xpad, xp, outdeg, W1)
    agga1, aggb1 = _scatter(ha1, hb1, src1d, dst1d, zfeat)
    ha2, hb2 = _tc2(agga1, aggb1, indeg, outdeg, b1r, ar, W2)
    agga2, aggb2 = _scatter(ha2, hb2, src1d, dst1d, zfeat)
    oa, ob = _tc3(agga2, aggb2, indeg, b2r, ar, Wm, bmr)

    return jnp.concatenate([oa[:N, 0], ob[:N, 0]], axis=0)
